# baseline (speedup 1.0000x reference)
import functools
import math

import jax
import jax.numpy as jnp
from jax import lax
from jax.experimental import pallas as pl
from jax.experimental.pallas import tpu as pltpu
from jax.experimental.pallas import tpu_sc as plsc

NC = 2
NS = 16
LANES = 16
CHUNK = 128
IB = 5


def _mesh():
    return plsc.VectorSubcoreMesh(core_axis_name="c", subcore_axis_name="s",
                                  num_cores=NC, num_subcores=NS)


def _sc_params():
    return pltpu.CompilerParams(needs_layout_passes=False,
                                use_tc_tiling_on_sc=False)


def _sc_degree(sdm, n_pad):
    ec = sdm.shape[0] - IB
    rpt = ec // (NC * NS)
    nbat = rpt // IB
    slc = n_pad // NS

    @functools.partial(
        pl.kernel,
        out_type=jax.ShapeDtypeStruct((NC * n_pad,), jnp.float32),
        mesh=_mesh(),
        scratch_types=[
            pltpu.VMEM((IB, 2, CHUNK), jnp.int32),
            pltpu.VMEM((CHUNK,), jnp.float32),
            pltpu.VMEM((slc,), jnp.float32),
            pltpu.VMEM_SHARED((n_pad,), jnp.float32),
            pltpu.SemaphoreType.DMA,
        ],
        compiler_params=_sc_params(),
    )
    def k(sdm_hbm, out_hbm, dbuf, obuf, zbuf, sacc, ssem):
        c = lax.axis_index("c")
        t = lax.axis_index("s")

        zeros16 = jnp.zeros((LANES,), jnp.float32)
        ones16 = jnp.ones((LANES,), jnp.float32)
        for g in range(CHUNK // LANES):
            obuf[pl.ds(g * LANES, LANES)] = ones16

        def zbody(i, carry):
            zbuf[pl.ds(i * LANES, LANES)] = zeros16
            return carry
        lax.fori_loop(0, slc // LANES, zbody, 0)
        base = t * slc
        pltpu.sync_copy(zbuf, sacc.at[pl.ds(base, slc)])
        plsc.subcore_barrier()

        row0 = (c * NS + t) * rpt

        def cbody(nb, carry):
            pltpu.sync_copy(sdm_hbm.at[pl.ds(row0 + nb * IB, IB)], dbuf)
            sds = [pltpu.async_copy(obuf, sacc.at[dbuf.at[kk, 1]],
                                    ssem, add=True)
                   for kk in range(IB)]
            for d in sds:
                d.wait()
            return carry
        lax.fori_loop(0, nbat, cbody, 0)

        plsc.subcore_barrier()
        pltpu.sync_copy(sacc.at[pl.ds(base, slc)],
                        out_hbm.at[pl.ds(c * n_pad + base, slc)])

    return k(sdm).reshape(NC, n_pad)


def _sc_scatter(y2, sdm, n, n_pad):
    ec = sdm.shape[0] - IB
    rpt = ec // NS
    nbat = rpt // IB
    slc = n_pad // NS
    nfull = slc // CHUNK
    tail = slc - nfull * CHUNK

    @functools.partial(
        pl.kernel,
        out_type=jax.ShapeDtypeStruct((NC * n_pad, 32), jnp.float32),
        mesh=_mesh(),
        scratch_types=[
            pltpu.VMEM((IB, 2, CHUNK), jnp.int32),
            pltpu.VMEM((IB, 2, CHUNK), jnp.int32),
            pltpu.VMEM((IB, CHUNK, 32), jnp.float32),
            pltpu.VMEM((CHUNK, 32), jnp.float32),
            pltpu.VMEM_SHARED((n_pad, 32), jnp.float32),
            pltpu.SemaphoreType.DMA,
            pltpu.SemaphoreType.DMA,
            pltpu.SemaphoreType.DMA,
        ],
        compiler_params=_sc_params(),
    )
    def k(y2_hbm, sdm_hbm, out_hbm,
          ebuf0, ebuf1, rows, zbuf, acc, gsem, ssem, isem):
        c = lax.axis_index("c")
        t = lax.axis_index("s")

        zeros16 = jnp.zeros((LANES,), jnp.float32)

        def zbody(i, carry):
            r = i // 2
            zbuf[r, pl.ds((i % 2) * LANES, LANES)] = zeros16
            return carry
        lax.fori_loop(0, CHUNK * 2, zbody, 0)

        base = t * slc
        for q in range(nfull):
            pltpu.sync_copy(zbuf, acc.at[pl.ds(base + q * CHUNK, CHUNK)])
        if tail:
            pltpu.sync_copy(zbuf.at[pl.ds(0, tail)],
                            acc.at[pl.ds(base + nfull * CHUNK, tail)])
        plsc.subcore_barrier()

        row0 = t * rpt
        coff = c * n

        def process(ebuf):
            for kk in range(IB):
                for gg in range(CHUNK // LANES):
                    sl = pl.ds(gg * LANES, LANES)
                    ebuf[kk, 0, sl] = ebuf[kk, 0, sl] + coff
            gds = [pltpu.async_copy(y2_hbm.at[ebuf.at[kk, 0]],
                                    rows.at[kk], gsem)
                   for kk in range(IB)]
            sds = []
            for kk in range(IB):
                gds[kk].wait()
                sds.append(pltpu.async_copy(rows.at[kk],
                                            acc.at[ebuf.at[kk, 1]],
                                            ssem, add=True))
            for d in sds:
                d.wait()

        def prefetch(nb, ebuf):
            r = row0 + nb * IB
            return pltpu.async_copy(sdm_hbm.at[pl.ds(r, IB)], ebuf, isem)

        pltpu.sync_copy(sdm_hbm.at[pl.ds(row0, IB)], ebuf0)

        def pair(i, carry):
            b0 = 2 * i
            pa = prefetch(b0 + 1, ebuf1)
            process(ebuf0)
            pa.wait()
            pb = prefetch(b0 + 2, ebuf0)
            process(ebuf1)
            pb.wait()
            return carry
        lax.fori_loop(0, nbat // 2, pair, 0)

        plsc.subcore_barrier()
        pltpu.sync_copy(acc.at[pl.ds(base, slc)],
                        out_hbm.at[pl.ds(c * n_pad + base, slc)])

    return k(y2, sdm)


def _pick_nb(n):
    for nb in (2000, 2500, 1000, 500, 250, 200, 125, 100, 50, 25, 16, 8):
        if n % nb == 0:
            return nb
    return n


def _tc_first(x, degp, w0):
    n, din = x.shape
    h = w0.shape[1]
    nb = _pick_nb(n)

    def body(x_ref, dp_ref, w_ref, y_ref, dinv_ref):
        deg = dp_ref[0] + dp_ref[1] + 1.0
        dinv = lax.rsqrt(deg)
        y = jnp.dot(x_ref[...], w_ref[...],
                    preferred_element_type=jnp.float32) * dinv
        y_ref[0] = y[:, :32]
        y_ref[1] = y[:, 32:]
        dinv_ref[...] = dinv

    return pl.pallas_call(
        body,
        grid=(n // nb,),
        in_specs=[
            pl.BlockSpec((nb, din), lambda i: (i, 0)),
            pl.BlockSpec((2, nb, 1), lambda i: (0, i, 0)),
            pl.BlockSpec((din, h), lambda i: (0, 0)),
        ],
        out_specs=[
            pl.BlockSpec((2, nb, 32), lambda i: (0, i, 0)),
            pl.BlockSpec((nb, 1), lambda i: (i, 0)),
        ],
        out_shape=[
            jax.ShapeDtypeStruct((2, n, 32), jnp.float32),
            jax.ShapeDtypeStruct((n, 1), jnp.float32),
        ],
    )(x, degp, w0)


def _tc_mid(agg3, y3, dinv, w, b_a, b_b):
    n = dinv.shape[0]
    h = w.shape[0]
    nb = _pick_nb(n)

    def body(ag, yy, dv, w_ref, ba, bb, oy):
        dinv = dv[...]
        ha = jnp.maximum((ag[0] + yy[0]) * dinv + ba[...], 0.0)
        hb = jnp.maximum((ag[1] + yy[1]) * dinv + bb[...], 0.0)
        hcat = jnp.concatenate([ha, hb], axis=1)
        y = jnp.dot(hcat, w_ref[...],
                    preferred_element_type=jnp.float32) * dinv
        oy[0] = y[:, :32]
        oy[1] = y[:, 32:]

    return pl.pallas_call(
        body,
        grid=(n // nb,),
        in_specs=[
            pl.BlockSpec((2, nb, 32), lambda i: (0, i, 0)),
            pl.BlockSpec((2, nb, 32), lambda i: (0, i, 0)),
            pl.BlockSpec((nb, 1), lambda i: (i, 0)),
            pl.BlockSpec((h, h), lambda i: (0, 0)),
            pl.BlockSpec((1, 32), lambda i: (0, 0)),
            pl.BlockSpec((1, 32), lambda i: (0, 0)),
        ],
        out_specs=pl.BlockSpec((2, nb, 32), lambda i: (0, i, 0)),
        out_shape=jax.ShapeDtypeStruct((2, n, 32), jnp.float32),
    )(agg3, y3, dinv, w, b_a, b_b)


def _tc_final(agg3, y3, dinv, b_a, b_b, wl, bl):
    n = dinv.shape[0]
    h = wl.shape[0]
    dout = wl.shape[1]
    nb = _pick_nb(n)

    def body(ag, yy, dv, ba, bb, w_ref, bl_ref, o):
        dinv = dv[...]
        ha = jnp.maximum((ag[0] + yy[0]) * dinv + ba[...], 0.0)
        hb = jnp.maximum((ag[1] + yy[1]) * dinv + bb[...], 0.0)
        hcat = jnp.concatenate([ha, hb], axis=1)
        o[...] = jnp.dot(hcat, w_ref[...],
                         preferred_element_type=jnp.float32) + bl_ref[...]

    return pl.pallas_call(
        body,
        grid=(n // nb,),
        in_specs=[
            pl.BlockSpec((2, nb, 32), lambda i: (0, i, 0)),
            pl.BlockSpec((2, nb, 32), lambda i: (0, i, 0)),
            pl.BlockSpec((nb, 1), lambda i: (i, 0)),
            pl.BlockSpec((1, 32), lambda i: (0, 0)),
            pl.BlockSpec((1, 32), lambda i: (0, 0)),
            pl.BlockSpec((h, dout), lambda i: (0, 0)),
            pl.BlockSpec((1, dout), lambda i: (0, 0)),
        ],
        out_specs=pl.BlockSpec((nb, dout), lambda i: (i, 0)),
        out_shape=jax.ShapeDtypeStruct((n, dout), jnp.float32),
    )(agg3, y3, dinv, b_a, b_b, wl, bl)


def kernel(x, edge_index, batch, W0, b0, W1, b1, W2, b2, Wl, bl):
    del batch
    n = x.shape[0]
    e = edge_index.shape[1]
    align = NS * LANES
    n_pad = ((n + align - 1) // align) * align
    if n_pad == n:
        n_pad += align

    grain = CHUNK * NC * NS * IB
    e_pad = ((e + grain - 1) // grain) * grain
    src = edge_index[0]
    dst = edge_index[1]
    src = jnp.concatenate(
        [src, jnp.zeros((e_pad - e + IB * CHUNK,), jnp.int32)])
    dst = jnp.concatenate(
        [dst, jnp.full((e_pad - e,), n, jnp.int32),
         jnp.zeros((IB * CHUNK,), jnp.int32)])
    sdm = jnp.stack([src.reshape(-1, CHUNK), dst.reshape(-1, CHUNK)], axis=1)

    degp = _sc_degree(sdm, n_pad)
    degp = degp.reshape(2, n_pad, 1)

    y3, dinv = _tc_first(x, degp, W0)

    b0a, b0b = b0[:32].reshape(1, 32), b0[32:].reshape(1, 32)
    b1a, b1b = b1[:32].reshape(1, 32), b1[32:].reshape(1, 32)
    b2a, b2b = b2[:32].reshape(1, 32), b2[32:].reshape(1, 32)
    bl2 = bl.reshape(1, -1)

    def scat(y3_):
        flat = _sc_scatter(y3_.reshape(2 * n, 32), sdm, n, n_pad)
        return flat.reshape(2, n_pad, 32)

    y3 = _tc_mid(scat(y3), y3, dinv, W1, b0a, b0b)
    y3 = _tc_mid(scat(y3), y3, dinv, W2, b1a, b1b)
    return _tc_final(scat(y3), y3, dinv, b2a, b2b, Wl, bl2)


# speedup vs baseline: 1.3957x; 1.3957x over previous
import functools
import math

import jax
import jax.numpy as jnp
from jax import lax
from jax.experimental import pallas as pl
from jax.experimental.pallas import tpu as pltpu
from jax.experimental.pallas import tpu_sc as plsc

NC = 2
NS = 16
LANES = 16
CHUNK = 120
IB = 6


def _mesh():
    return plsc.VectorSubcoreMesh(core_axis_name="c", subcore_axis_name="s",
                                  num_cores=NC, num_subcores=NS)


def _sc_params():
    return pltpu.CompilerParams(needs_layout_passes=False,
                                use_tc_tiling_on_sc=False)


def _sc_degree(sdm, n_pad):
    ec = sdm.shape[0] - IB
    rpt = ec // (NC * NS)
    nbat = rpt // IB
    slc = n_pad // NS

    @functools.partial(
        pl.kernel,
        out_type=jax.ShapeDtypeStruct((NC * n_pad,), jnp.float32),
        mesh=_mesh(),
        scratch_types=[
            pltpu.VMEM((IB, 2, CHUNK), jnp.int32),
            pltpu.VMEM((CHUNK,), jnp.float32),
            pltpu.VMEM((slc,), jnp.float32),
            pltpu.VMEM_SHARED((n_pad,), jnp.float32),
            pltpu.SemaphoreType.DMA,
        ],
        compiler_params=_sc_params(),
    )
    def k(sdm_hbm, out_hbm, dbuf, obuf, zbuf, sacc, ssem):
        c = lax.axis_index("c")
        t = lax.axis_index("s")

        zeros16 = jnp.zeros((LANES,), jnp.float32)
        ones16 = jnp.ones((LANES,), jnp.float32)
        for g in range(CHUNK // LANES):
            obuf[pl.ds(g * LANES, LANES)] = ones16

        def zbody(i, carry):
            zbuf[pl.ds(i * LANES, LANES)] = zeros16
            return carry
        lax.fori_loop(0, slc // LANES, zbody, 0)
        base = t * slc
        pltpu.sync_copy(zbuf, sacc.at[pl.ds(base, slc)])
        plsc.subcore_barrier()

        row0 = (c * NS + t) * rpt

        def cbody(nb, carry):
            pltpu.sync_copy(sdm_hbm.at[pl.ds(row0 + nb * IB, IB)], dbuf)
            sds = [pltpu.async_copy(obuf, sacc.at[dbuf.at[kk, 1]],
                                    ssem, add=True)
                   for kk in range(IB)]
            for d in sds:
                d.wait()
            return carry
        lax.fori_loop(0, nbat, cbody, 0)

        plsc.subcore_barrier()
        pltpu.sync_copy(sacc.at[pl.ds(base, slc)],
                        out_hbm.at[pl.ds(c * n_pad + base, slc)])

    return k(sdm).reshape(NC, n_pad)


def _sc_scatter(y2, sdm, n, n_pad):
    ec = sdm.shape[0] - IB
    rpt = ec // NS
    nbat = rpt // IB
    slc = n_pad // NS
    nfull = slc // CHUNK
    tail = slc - nfull * CHUNK

    @functools.partial(
        pl.kernel,
        out_type=jax.ShapeDtypeStruct((NC * n_pad, 32), jnp.float32),
        mesh=_mesh(),
        scratch_types=[
            pltpu.VMEM((IB, 2, CHUNK), jnp.int32),
            pltpu.VMEM((IB, 2, CHUNK), jnp.int32),
            pltpu.VMEM((IB, CHUNK, 32), jnp.float32),
            pltpu.VMEM((CHUNK, 32), jnp.float32),
            pltpu.VMEM_SHARED((n_pad, 32), jnp.float32),
            pltpu.SemaphoreType.DMA,
            pltpu.SemaphoreType.DMA,
            pltpu.SemaphoreType.DMA,
        ],
        compiler_params=_sc_params(),
    )
    def k(y2_hbm, sdm_hbm, out_hbm,
          ebuf0, ebuf1, rows, zbuf, acc, gsem, ssem, isem):
        c = lax.axis_index("c")
        t = lax.axis_index("s")

        zeros16 = jnp.zeros((LANES,), jnp.float32)

        def zbody(i, carry):
            r = i // 2
            zbuf[r, pl.ds((i % 2) * LANES, LANES)] = zeros16
            return carry
        lax.fori_loop(0, CHUNK * 2, zbody, 0)

        base = t * slc
        for q in range(nfull):
            pltpu.sync_copy(zbuf, acc.at[pl.ds(base + q * CHUNK, CHUNK)])
        if tail:
            pltpu.sync_copy(zbuf.at[pl.ds(0, tail)],
                            acc.at[pl.ds(base + nfull * CHUNK, tail)])
        plsc.subcore_barrier()

        row0 = t * rpt
        coff = c * n

        def process(ebuf):
            for kk in range(IB):
                for gg in range(CHUNK // LANES):
                    sl = pl.ds(gg * LANES, LANES)
                    ebuf[kk, 0, sl] = ebuf[kk, 0, sl] + coff
            gds = [pltpu.async_copy(y2_hbm.at[ebuf.at[kk, 0]],
                                    rows.at[kk], gsem)
                   for kk in range(IB)]
            sds = []
            for kk in range(IB):
                gds[kk].wait()
                sds.append(pltpu.async_copy(rows.at[kk],
                                            acc.at[ebuf.at[kk, 1]],
                                            ssem, add=True))
            for d in sds:
                d.wait()

        def prefetch(nb, ebuf):
            r = row0 + nb * IB
            return pltpu.async_copy(sdm_hbm.at[pl.ds(r, IB)], ebuf, isem)

        pltpu.sync_copy(sdm_hbm.at[pl.ds(row0, IB)], ebuf0)

        def pair(i, carry):
            b0 = 2 * i
            pa = prefetch(b0 + 1, ebuf1)
            process(ebuf0)
            pa.wait()
            pb = prefetch(b0 + 2, ebuf0)
            process(ebuf1)
            pb.wait()
            return carry
        lax.fori_loop(0, nbat // 2, pair, 0)

        plsc.subcore_barrier()
        pltpu.sync_copy(acc.at[pl.ds(base, slc)],
                        out_hbm.at[pl.ds(c * n_pad + base, slc)])

    return k(y2, sdm)


def _pick_nb(n):
    for nb in (2000, 2500, 1000, 500, 250, 200, 125, 100, 50, 25, 16, 8):
        if n % nb == 0:
            return nb
    return n


def _tc_first(x, degp, w0):
    n, din = x.shape
    h = w0.shape[1]
    nb = _pick_nb(n)

    def body(x_ref, dp_ref, w_ref, y_ref, dinv_ref):
        deg = dp_ref[0] + dp_ref[1] + 1.0
        dinv = lax.rsqrt(deg)
        y = jnp.dot(x_ref[...], w_ref[...],
                    preferred_element_type=jnp.float32) * dinv
        y_ref[0] = y[:, :32]
        y_ref[1] = y[:, 32:]
        dinv_ref[...] = dinv

    return pl.pallas_call(
        body,
        grid=(n // nb,),
        in_specs=[
            pl.BlockSpec((nb, din), lambda i: (i, 0)),
            pl.BlockSpec((2, nb, 1), lambda i: (0, i, 0)),
            pl.BlockSpec((din, h), lambda i: (0, 0)),
        ],
        out_specs=[
            pl.BlockSpec((2, nb, 32), lambda i: (0, i, 0)),
            pl.BlockSpec((nb, 1), lambda i: (i, 0)),
        ],
        out_shape=[
            jax.ShapeDtypeStruct((2, n, 32), jnp.float32),
            jax.ShapeDtypeStruct((n, 1), jnp.float32),
        ],
    )(x, degp, w0)


def _tc_mid(agg3, y3, dinv, w, b_a, b_b):
    n = dinv.shape[0]
    h = w.shape[0]
    nb = _pick_nb(n)

    def body(ag, yy, dv, w_ref, ba, bb, oy):
        dinv = dv[...]
        ha = jnp.maximum((ag[0] + yy[0]) * dinv + ba[...], 0.0)
        hb = jnp.maximum((ag[1] + yy[1]) * dinv + bb[...], 0.0)
        hcat = jnp.concatenate([ha, hb], axis=1)
        y = jnp.dot(hcat, w_ref[...],
                    preferred_element_type=jnp.float32) * dinv
        oy[0] = y[:, :32]
        oy[1] = y[:, 32:]

    return pl.pallas_call(
        body,
        grid=(n // nb,),
        in_specs=[
            pl.BlockSpec((2, nb, 32), lambda i: (0, i, 0)),
            pl.BlockSpec((2, nb, 32), lambda i: (0, i, 0)),
            pl.BlockSpec((nb, 1), lambda i: (i, 0)),
            pl.BlockSpec((h, h), lambda i: (0, 0)),
            pl.BlockSpec((1, 32), lambda i: (0, 0)),
            pl.BlockSpec((1, 32), lambda i: (0, 0)),
        ],
        out_specs=pl.BlockSpec((2, nb, 32), lambda i: (0, i, 0)),
        out_shape=jax.ShapeDtypeStruct((2, n, 32), jnp.float32),
    )(agg3, y3, dinv, w, b_a, b_b)


def _tc_final(agg3, y3, dinv, b_a, b_b, wl, bl):
    n = dinv.shape[0]
    h = wl.shape[0]
    dout = wl.shape[1]
    nb = _pick_nb(n)

    def body(ag, yy, dv, ba, bb, w_ref, bl_ref, o):
        dinv = dv[...]
        ha = jnp.maximum((ag[0] + yy[0]) * dinv + ba[...], 0.0)
        hb = jnp.maximum((ag[1] + yy[1]) * dinv + bb[...], 0.0)
        hcat = jnp.concatenate([ha, hb], axis=1)
        o[...] = jnp.dot(hcat, w_ref[...],
                         preferred_element_type=jnp.float32) + bl_ref[...]

    return pl.pallas_call(
        body,
        grid=(n // nb,),
        in_specs=[
            pl.BlockSpec((2, nb, 32), lambda i: (0, i, 0)),
            pl.BlockSpec((2, nb, 32), lambda i: (0, i, 0)),
            pl.BlockSpec((nb, 1), lambda i: (i, 0)),
            pl.BlockSpec((1, 32), lambda i: (0, 0)),
            pl.BlockSpec((1, 32), lambda i: (0, 0)),
            pl.BlockSpec((h, dout), lambda i: (0, 0)),
            pl.BlockSpec((1, dout), lambda i: (0, 0)),
        ],
        out_specs=pl.BlockSpec((nb, dout), lambda i: (i, 0)),
        out_shape=jax.ShapeDtypeStruct((n, dout), jnp.float32),
    )(agg3, y3, dinv, b_a, b_b, wl, bl)


def kernel(x, edge_index, batch, W0, b0, W1, b1, W2, b2, Wl, bl):
    del batch
    n = x.shape[0]
    e = edge_index.shape[1]
    align = NS * LANES
    n_pad = ((n + align - 1) // align) * align
    if n_pad == n:
        n_pad += align

    grain = CHUNK * NC * NS * IB
    e_pad = ((e + grain - 1) // grain) * grain
    src = edge_index[0]
    dst = edge_index[1]
    src = jnp.concatenate(
        [src, jnp.zeros((e_pad - e + IB * CHUNK,), jnp.int32)])
    dst = jnp.concatenate(
        [dst, jnp.full((e_pad - e,), n, jnp.int32),
         jnp.zeros((IB * CHUNK,), jnp.int32)])
    sdm = jnp.stack([src.reshape(-1, CHUNK), dst.reshape(-1, CHUNK)], axis=1)

    degp = _sc_degree(sdm, n_pad)
    degp = degp.reshape(2, n_pad, 1)

    y3, dinv = _tc_first(x, degp, W0)

    b0a, b0b = b0[:32].reshape(1, 32), b0[32:].reshape(1, 32)
    b1a, b1b = b1[:32].reshape(1, 32), b1[32:].reshape(1, 32)
    b2a, b2b = b2[:32].reshape(1, 32), b2[32:].reshape(1, 32)
    bl2 = bl.reshape(1, -1)

    def scat(y3_):
        flat = _sc_scatter(y3_.reshape(2 * n, 32), sdm, n, n_pad)
        return flat.reshape(2, n_pad, 32)

    y3 = _tc_mid(scat(y3), y3, dinv, W1, b0a, b0b)
    y3 = _tc_mid(scat(y3), y3, dinv, W2, b1a, b1b)
    return _tc_final(scat(y3), y3, dinv, b2a, b2b, Wl, bl2)
